# Initial kernel scaffold; baseline (speedup 1.0000x reference)
import jax
import jax.numpy as jnp
from jax.experimental import pallas as pl


def kernel(X):
    raise NotImplementedError("write your pallas kernel here")


# speedup vs baseline: 3.6148x; 3.6148x over previous
import functools

import jax
import jax.numpy as jnp
from jax import lax
from jax.experimental import pallas as pl
from jax.experimental.pallas import tpu as pltpu
from jax.experimental.pallas import tpu_sc as plsc

_NB_ACTIVE = 16
_LANES = 16
_NC = 2
_NS = 16
_NW = _NC * _NS


def _sort16(v):
    return plsc.sort_key_val(v, v)[0]


def _merge16(a, b):
    return _sort16(jnp.maximum(a, lax.rev(b, (0,))))


def _row_topk_threshold(buf, base, n_chunks):
    def go(lo, hi):
        if hi - lo == 1:
            return _sort16(buf[pl.ds(base + _LANES * lo, _LANES)])
        mid = (lo + hi) // 2
        return _merge16(go(lo, mid), go(mid, hi))

    return jnp.min(go(0, n_chunks))


def _wta_body(n_rows, row_n, block_rows, x_hbm, out_hbm, buf):
    n_chunks = row_n // _LANES
    rows_per_w = n_rows // _NW
    n_blocks = rows_per_w // block_rows
    blk_elems = block_rows * row_n

    wid = lax.axis_index("s") * _NC + lax.axis_index("c")
    w_base = wid * rows_per_w * row_n

    @pl.loop(0, n_blocks)
    def _blocks(blk):
        hbm_off = w_base + blk * blk_elems
        pltpu.sync_copy(x_hbm.at[pl.ds(hbm_off, blk_elems)], buf)

        @pl.loop(0, block_rows)
        def _rows(r):
            base = r * row_n
            t = _row_topk_threshold(buf, base, n_chunks)
            for c in range(n_chunks):
                sl = pl.ds(base + _LANES * c, _LANES)
                v = buf[sl]
                buf[sl] = jnp.where(v >= t, v, 0.0)

        pltpu.sync_copy(buf, out_hbm.at[pl.ds(hbm_off, blk_elems)])


def kernel(X):
    B, C, H, W = X.shape
    n = H * W
    rows = B * C
    block_rows = 16

    mesh = plsc.VectorSubcoreMesh(
        core_axis_name="c", subcore_axis_name="s",
        num_cores=_NC, num_subcores=_NS)

    body = functools.partial(_wta_body, rows, n, block_rows)
    out = pl.kernel(
        body,
        out_type=jax.ShapeDtypeStruct((rows * n,), jnp.float32),
        mesh=mesh,
        compiler_params=pltpu.CompilerParams(needs_layout_passes=False),
        scratch_types=[pltpu.VMEM((block_rows * n,), jnp.float32)],
    )(X.reshape(rows * n))
    return out.reshape(B, C, H, W)
